# baseline (speedup 1.0000x reference)
import jax
import jax.numpy as jnp
from jax import lax
from jax.experimental import pallas as pl
from jax.experimental.pallas import tpu as pltpu
from jax.experimental.pallas import tpu_sc as plsc

_B, _Q, _W = 4096, 50, 16
_DW, _DC = 64, 16
_DWP = 128
_VC = 512
_NW = 32
_BPW = _B // _NW
_CH = 64
_NT = (_DC + _DW) // 8
_R = 4
_NCH = _Q * (_BPW // _CH)
_NITER = _NCH // _R
_L = 16


def _splat(v, dtype=jnp.int32):
    return jnp.full((_L,), v, dtype)


def _sc_body(qw_hbm, qc_hbm, qlen_hbm, wtab_hbm, ctab_hbm, out_hbm,
             ctab_v, len_v, widx_v, cidx_v, wbuf_v, wp_v, obuf_v,
             sem_g, sem_ci, sem_o):
    nc = 2
    wid = lax.axis_index("s") * nc + lax.axis_index("c")
    b0 = wid * _BPW

    pltpu.sync_copy(ctab_hbm, ctab_v)
    pltpu.sync_copy(qlen_hbm.at[pl.ds(b0, _BPW)], len_v)
    pltpu.sync_copy(qw_hbm.at[:, pl.ds(b0, _BPW)], widx_v)

    lane = lax.iota(jnp.int32, _L)

    def fire_inputs(q, boff, j):
        pltpu.async_copy(qc_hbm.at[q, :, pl.ds(b0 + boff, _CH)], cidx_v.at[j],
                         sem_ci.at[j])
        pltpu.async_copy(wtab_hbm.at[widx_v.at[q, pl.ds(boff, _CH)]],
                         wbuf_v.at[j], sem_g.at[j])

    def drain_inputs(j):
        pltpu.make_async_copy(qc_hbm.at[0, :, pl.ds(b0, _CH)], cidx_v.at[j],
                              sem_ci.at[j]).wait()
        pltpu.make_async_copy(wtab_hbm.at[pl.ds(0, _CH), :],
                              wbuf_v.at[j], sem_g.at[j]).wait()

    def fire_out(q, boff, j):
        pltpu.async_copy(obuf_v.at[j],
                         out_hbm.at[q, :, wid, :, pl.ds(boff, _CH)],
                         sem_o.at[j])

    def drain_out(j):
        pltpu.make_async_copy(obuf_v.at[j],
                              out_hbm.at[0, :, wid, :, pl.ds(0, _CH)],
                              sem_o.at[j]).wait()

    def compute_chunk(q, boff, cidx, wbuf, obuf):
        qv = jnp.full((_L,), q, jnp.int32)

        def pos_body(p4, carry):
            for u in range(4):
                p = p4 * 4 + u
                for k in range(_DW // _L):
                    wp_v[p, pl.ds(k * _L, _L)] = wbuf[p, pl.ds(k * _L, _L)]
            return carry

        lax.fori_loop(0, _CH // 4, pos_body, 0)

        def group_body(g, carry):
            goff = g * _L
            lenv = len_v[pl.ds(boff + goff, _L)]
            valid = qv < lenv
            maskf = jnp.where(valid, 1.0, 0.0).astype(jnp.float32)

            acc = [None] * _DC
            for w in range(_W):
                cw = cidx[w, pl.ds(goff, _L)]
                for d in range(_DC):
                    v = plsc.load_gather(ctab_v, [_splat(d), cw])
                    acc[d] = v if acc[d] is None else jnp.maximum(acc[d], v)
            for d in range(_DC):
                obuf[d // 8, d % 8, pl.ds(goff, _L)] = acc[d] * maskf

            b_rel = goff + lane
            for dd in range(_DW):
                v = plsc.load_gather(wp_v, [b_rel, _splat(dd)])
                dout = _DC + dd
                obuf[dout // 8, dout % 8, pl.ds(goff, _L)] = v * maskf
            return carry

        lax.fori_loop(0, _CH // _L, group_body, 0)

    fire_inputs(0, 0, 0)

    def iter_body(i, carry):
        for j in range(_R):
            c = i * _R + j
            q = i * 2 + (j // 2)
            boff = (j % 2) * _CH
            nq = q + (j % 2)
            nboff = ((j + 1) % 2) * _CH
            nj = (j + 1) % _R
            if j == _R - 1:
                drain_out(nj)
                @pl.when(i < _NITER - 1)
                def _fire():
                    fire_inputs(nq, nboff, nj)
            else:
                @pl.when(i > 0)
                def _drain():
                    drain_out(nj)
                fire_inputs(nq, nboff, nj)
            drain_inputs(j)
            compute_chunk(q, boff, cidx_v.at[j], wbuf_v.at[j], obuf_v.at[j])
            fire_out(q, boff, j)
        return carry

    lax.fori_loop(0, _NITER, iter_body, 0)

    for j in range(1, _R):
        drain_out(j)


_sc_call = pl.kernel(
    _sc_body,
    out_type=jax.ShapeDtypeStruct((_Q, _NT, _NW, 8, _BPW), jnp.float32),
    mesh=plsc.VectorSubcoreMesh(core_axis_name="c", subcore_axis_name="s"),
    scratch_types=[
        pltpu.VMEM((_DC, _VC), jnp.float32),
        pltpu.VMEM((_BPW,), jnp.int32),
        pltpu.VMEM((_Q, _BPW), jnp.int32),
        pltpu.VMEM((_R, _W, _CH), jnp.int32),
        pltpu.VMEM((_R, _CH, _DWP), jnp.float32),
        pltpu.VMEM((_CH, _DW + 1), jnp.float32),
        pltpu.VMEM((_R, _NT, 8, _CH), jnp.float32),
        pltpu.SemaphoreType.DMA((_R,)),
        pltpu.SemaphoreType.DMA((_R,)),
        pltpu.SemaphoreType.DMA((_R,)),
    ],
    compiler_params=pltpu.CompilerParams(use_tc_tiling_on_sc=False,
                                         needs_layout_passes=False),
)


def kernel(question_words, question_chars, question_len, word_table, char_table):
    qw_t = question_words.astype(jnp.int32).T
    qc_t = question_chars.astype(jnp.int32).transpose(1, 2, 0)
    ctab_t = char_table.T
    ql = question_len.astype(jnp.int32)
    wt128 = jnp.pad(word_table, ((0, 0), (0, _DWP - _DW)))
    out5 = _sc_call(qw_t, qc_t, ql, wt128, ctab_t)
    return out5.transpose(2, 4, 0, 1, 3).reshape(_B, _Q, _DC + _DW)
